# baseline (speedup 1.0000x reference)
import functools

import jax
import jax.numpy as jnp
from jax import lax
from jax.experimental import pallas as pl

B = 2
NPTS = 4096
N = B * NPTS
NROIS = 64
GRID = 3
C = 32
CROI = 64
NS = 32
PNS = 16
RADIUS = 0.8
TEMP = 1.0
DIV = 2.0
MIN_R = 0.01
M = B * NROIS * GRID**3
MB = NROIS * GRID**3
MBP = 1792
MP = B * MBP
PW = 48
QBLK = 128
NBB = MBP // QBLK
NBLK = B * NBB
PCH = 512


def _sel_gather(nxq, r2q, xyzb, p2b, ptabb, ns):
  def f(nx_ref, r2_ref, xyz_ref, p2_ref, pt_ref, rows_ref, emp_ref):
    nx = nx_ref[...]
    r2 = r2_ref[...]
    pxyz = xyz_ref[0]
    dx = nx[:, 0:1] - pxyz[0:1, :]
    dy = nx[:, 1:2] - pxyz[1:2, :]
    dz = nx[:, 2:3] - pxyz[2:3, :]
    d2 = dx * dx + dy * dy + dz * dz
    vm = (d2 < r2).astype(jnp.float32)
    rio = lax.broadcasted_iota(jnp.int32, (PCH, PCH), 0)
    cio = lax.broadcasted_iota(jnp.int32, (PCH, PCH), 1)
    lt = (rio < cio).astype(jnp.float32)
    carry = jnp.zeros((QBLK, 1), jnp.float32)
    ranks = []
    for c in range(NPTS // PCH):
      vmc = vm[:, c * PCH:(c + 1) * PCH]
      ranks.append(jnp.dot(vmc, lt, preferred_element_type=jnp.float32) + carry)
      carry = carry + jnp.sum(vmc, axis=1, keepdims=True)
    rank = jnp.concatenate(ranks, axis=1)
    cnt = carry
    ptab = pt_ref[0]
    gs = []
    g0 = None
    for s in range(ns):
      sel = vm * (rank == float(s)).astype(jnp.float32)
      g = jnp.dot(sel, ptab, preferred_element_type=jnp.float32,
                  precision=lax.Precision.HIGHEST)
      if s == 0:
        g0 = g
      else:
        g = g + jnp.where(cnt <= float(s), 1.0, 0.0) * g0
      gs.append(g.reshape(QBLK, 1, PW))
    g3 = jnp.concatenate(gs, axis=1)
    rows_ref[...] = g3.reshape(QBLK * ns, PW)
    emp_ref[...] = jnp.where(cnt == 0.0, 1.0, 0.0)

  return pl.pallas_call(
      f, grid=(NBLK,),
      in_specs=[
          pl.BlockSpec((QBLK, 3), lambda j: (j, 0)),
          pl.BlockSpec((QBLK, 1), lambda j: (j, 0)),
          pl.BlockSpec((1, 3, NPTS), lambda j: (j // NBB, 0, 0)),
          pl.BlockSpec((1, 1, NPTS), lambda j: (j // NBB, 0, 0)),
          pl.BlockSpec((1, NPTS, PW), lambda j: (j // NBB, 0, 0)),
      ],
      out_specs=[
          pl.BlockSpec((QBLK * ns, PW), lambda j: (j, 0)),
          pl.BlockSpec((QBLK, 1), lambda j: (j, 0)),
      ],
      out_shape=[
          jax.ShapeDtypeStruct((MP * ns, PW), jnp.float32),
          jax.ShapeDtypeStruct((MP, 1), jnp.float32),
      ],
  )(nxq, r2q, xyzb, p2b, ptabb)


def _y3(rows_ref, nx_ref, emp_ref, w_ref, ns, co):
  rows3 = rows_ref[...].reshape(QBLK, ns, PW)
  gxyz = rows3[:, :, 0:3] - nx_ref[...][:, None, :]
  g3 = jnp.concatenate([gxyz, rows3[:, :, 3:]], axis=2)
  y2 = jnp.dot(g3.reshape(QBLK * ns, PW), w_ref[...].T,
               preferred_element_type=jnp.float32)
  y3 = y2.reshape(QBLK, ns, co)
  return jnp.where(emp_ref[...] > 0, 0.0, y3)


def _bn(y, st_ref, g_ref, b_ref, count, co):
  mean = (st_ref[0:1, 0:co] / count).reshape(1, 1, co)
  var = (st_ref[1:2, 0:co] / count).reshape(1, 1, co) - mean * mean
  inv = lax.rsqrt(var + 1e-5)
  z = (y - mean) * inv * g_ref[...].reshape(1, 1, co) + b_ref[...].reshape(1, 1, co)
  return jnp.maximum(z, 0.0)


def _stats_update(out_ref, y, co, j):
  @pl.when(j == 0)
  def _():
    out_ref[...] = jnp.zeros_like(out_ref)
  out_ref[0:1, 0:co] += jnp.sum(y, axis=(0, 1)).reshape(1, co)
  out_ref[1:2, 0:co] += jnp.sum(y * y, axis=(0, 1)).reshape(1, co)


def _specs(ns, co_w):
  rows_s = pl.BlockSpec((QBLK * ns, PW), lambda j: (j, 0))
  nx_s = pl.BlockSpec((QBLK, 3), lambda j: (j, 0))
  emp_s = pl.BlockSpec((QBLK, 1, 1), lambda j: (j, 0, 0))
  w_s = pl.BlockSpec((co_w, PW), lambda j: (0, 0))
  st_s = pl.BlockSpec((8, 128), lambda j: (0, 0))
  return rows_s, nx_s, emp_s, w_s, st_s


def _stats1(rows, nx, emp3, wp, ns, co):
  rows_s, nx_s, emp_s, w_s, st_s = _specs(ns, co)

  def f(rows_ref, nx_ref, emp_ref, w_ref, out_ref):
    j = pl.program_id(0)
    y = _y3(rows_ref, nx_ref, emp_ref, w_ref, ns, co)
    _stats_update(out_ref, y, co, j)

  return pl.pallas_call(
      f, grid=(NBLK,),
      in_specs=[rows_s, nx_s, emp_s, w_s],
      out_specs=st_s,
      out_shape=jax.ShapeDtypeStruct((8, 128), jnp.float32),
  )(rows, nx, emp3, wp)


def _apply1(rows, nx, emp3, wp, g, b, st):
  rows_s, nx_s, emp_s, w_s, st_s = _specs(PNS, 16)
  gb_s = pl.BlockSpec((1, 16), lambda j: (0, 0))

  def f(rows_ref, nx_ref, emp_ref, w_ref, g_ref, b_ref, st_ref, out_ref):
    y = _y3(rows_ref, nx_ref, emp_ref, w_ref, PNS, 16)
    z = _bn(y, st_ref, g_ref, b_ref, float(M * PNS), 16)
    out_ref[...] = jnp.max(z, axis=1)

  return pl.pallas_call(
      f, grid=(NBLK,),
      in_specs=[rows_s, nx_s, emp_s, w_s, gb_s, gb_s, st_s],
      out_specs=pl.BlockSpec((QBLK, 16), lambda j: (j, 0)),
      out_shape=jax.ShapeDtypeStruct((MP, 16), jnp.float32),
  )(rows, nx, emp3, wp, g, b, st)


def _bfr(x):
  xi = lax.bitcast_convert_type(x, jnp.uint32)
  xr = (xi + jnp.uint32(0x7FFF) + ((xi >> 16) & jnp.uint32(1))) & jnp.uint32(0xFFFF0000)
  return lax.bitcast_convert_type(xr, jnp.float32)


def _fc(pf, fwrep, rf, w2, bias):
  def f(pf_ref, fw_ref, rf_ref, w2_ref, b_ref, out_ref):
    a = jnp.sum(_bfr(pf_ref[...]) * _bfr(fw_ref[...]), axis=1, keepdims=True)
    mio = lax.broadcasted_iota(jnp.int32, (B * NROIS, MP), 1)
    rio = lax.broadcasted_iota(jnp.int32, (B * NROIS, MP), 0)
    roi_of = (mio // MBP) * NROIS + (mio % MBP) // (GRID**3)
    vrow = (mio % MBP) < MB
    S = ((roi_of == rio) & vrow).astype(jnp.float32)
    resid = jnp.dot(S, a, preferred_element_type=jnp.float32,
                    precision=lax.Precision.HIGHEST)
    resid += jnp.dot(_bfr(rf_ref[...]), _bfr(w2_ref[...]),
                     preferred_element_type=jnp.float32,
                     precision=lax.Precision.HIGHEST)
    r = jnp.maximum((resid + b_ref[0, 0]) / DIV + RADIUS, MIN_R)
    qio = lax.broadcasted_iota(jnp.int32, (MP, B * NROIS), 0)
    cio = lax.broadcasted_iota(jnp.int32, (MP, B * NROIS), 1)
    roi_oq = (qio // MBP) * NROIS + (qio % MBP) // (GRID**3)
    vq = (qio % MBP) < MB
    T = ((roi_oq == cio) & vq).astype(jnp.float32)
    r2q = jnp.dot(T, r * r, preferred_element_type=jnp.float32,
                  precision=lax.Precision.HIGHEST)
    qv = lax.broadcasted_iota(jnp.int32, (MP, 1), 0)
    out_ref[...] = jnp.where(qv % MBP < MB, r2q, -1.0)

  def full(s):
    return pl.BlockSpec(s, lambda: tuple(0 for _ in s))

  return pl.pallas_call(
      f,
      in_specs=[full((MP, 16)), full((MP, 16)), full((B * NROIS, CROI)),
                full((CROI, 1)), full((1, 1))],
      out_specs=full((MP, 1)),
      out_shape=jax.ShapeDtypeStruct((MP, 1), jnp.float32),
  )(pf, fwrep, rf, w2, bias)


def _stats2(rows, nx, emp3, w1p, g1, b1, st1, w2):
  rows_s, nx_s, emp_s, w_s, st_s = _specs(NS, 32)
  gb_s = pl.BlockSpec((1, 32), lambda j: (0, 0))
  w2_s = pl.BlockSpec((64, 32), lambda j: (0, 0))

  def f(rows_ref, nx_ref, emp_ref, w1_ref, g1_ref, b1_ref, st1_ref, w2_ref,
        out_ref):
    j = pl.program_id(0)
    y1 = _y3(rows_ref, nx_ref, emp_ref, w1_ref, NS, 32)
    z1 = _bn(y1, st1_ref, g1_ref, b1_ref, float(M * NS), 32)
    y2 = jnp.dot(z1.reshape(QBLK * NS, 32), w2_ref[...].T,
                 preferred_element_type=jnp.float32)
    y2 = y2.reshape(QBLK, NS, 64)
    _stats_update(out_ref, y2, 64, j)

  return pl.pallas_call(
      f, grid=(NBLK,),
      in_specs=[rows_s, nx_s, emp_s, w_s, gb_s, gb_s, st_s, w2_s],
      out_specs=st_s,
      out_shape=jax.ShapeDtypeStruct((8, 128), jnp.float32),
  )(rows, nx, emp3, w1p, g1, b1, st1, w2)


def _apply2(rows, nx, emp3, w1p, g1, b1, st1, w2, g2, b2, st2, tpd):
  rows_s, nx_s, emp_s, w_s, st_s = _specs(NS, 32)
  gb1_s = pl.BlockSpec((1, 32), lambda j: (0, 0))
  gb2_s = pl.BlockSpec((1, 64), lambda j: (0, 0))
  w2_s = pl.BlockSpec((64, 32), lambda j: (0, 0))
  tpd_s = pl.BlockSpec((1, 1), lambda j: (0, 0))

  def f(rows_ref, nx_ref, emp_ref, w1_ref, g1_ref, b1_ref, st1_ref, w2_ref,
        g2_ref, b2_ref, st2_ref, tpd_ref, out_ref):
    y1 = _y3(rows_ref, nx_ref, emp_ref, w1_ref, NS, 32)
    z1 = _bn(y1, st1_ref, g1_ref, b1_ref, float(M * NS), 32)
    y2 = jnp.dot(z1.reshape(QBLK * NS, 32), w2_ref[...].T,
                 preferred_element_type=jnp.float32).reshape(QBLK, NS, 64)
    z2 = _bn(y2, st2_ref, g2_ref, b2_ref, float(M * NS), 64)
    rows3 = rows_ref[...].reshape(QBLK, NS, PW)
    gxyz = rows3[:, :, 0:3] - nx_ref[...][:, None, :]
    d2v = jnp.sum(gxyz * gxyz, axis=2, keepdims=True)
    dist = jnp.sqrt(d2v + 1e-12)
    logit = -dist / (TEMP * tpd_ref[0, 0])
    mx = jnp.max(logit, axis=1, keepdims=True)
    e = jnp.exp(logit - mx)
    w = e / jnp.sum(e, axis=1, keepdims=True)
    w = jnp.where(emp_ref[...] > 0, 0.0, w)
    out_ref[...] = jnp.max(w * z2, axis=1)

  return pl.pallas_call(
      f, grid=(NBLK,),
      in_specs=[rows_s, nx_s, emp_s, w_s, gb1_s, gb1_s, st_s, w2_s,
                gb2_s, gb2_s, st_s, tpd_s],
      out_specs=pl.BlockSpec((QBLK, 64), lambda j: (j, 0)),
      out_shape=jax.ShapeDtypeStruct((MP, 64), jnp.float32),
  )(rows, nx, emp3, w1p, g1, b1, st1, w2, g2, b2, st2, tpd)


def _padw(w):
  return jnp.concatenate(
      [w, jnp.zeros((w.shape[0], PW - w.shape[1]), jnp.float32)], axis=1)


def _pad_q(x):
  d = x.shape[1]
  xb = x.reshape(B, MB, d)
  pad = jnp.zeros((B, MBP - MB, d), x.dtype)
  return jnp.concatenate([xb, pad], axis=1).reshape(MP, d)


def kernel(xyz, xyz_batch_cnt, rois, roi_features, features,
           temperature_decay, pmlp_W, pmlp_gamma, pmlp_beta, fc_W, fc_b,
           fmlp_W1, fmlp_g1, fmlp_b1, fmlp_W2, fmlp_g2, fmlp_b2):
  new_xyz = rois.reshape(-1, 3)
  nxq = _pad_q(new_xyz)
  xyzb = xyz.reshape(B, NPTS, 3).transpose(0, 2, 1)
  p2b = jnp.sum(xyzb * xyzb, axis=1, keepdims=True)
  ptabb = jnp.concatenate(
      [xyz, features, jnp.zeros((N, PW - 3 - C), jnp.float32)],
      axis=1).reshape(B, NPTS, PW)
  qv = jnp.arange(MP)
  r2p1 = jnp.where(qv % MBP < MB, jnp.float32(RADIUS**2), -1.0).reshape(MP, 1)

  rows1, emp1 = _sel_gather(nxq, r2p1, xyzb, p2b, ptabb, PNS)
  emp3_1 = emp1.reshape(MP, 1, 1)
  wp1 = _padw(pmlp_W)
  st1 = _stats1(rows1, nxq, emp3_1, wp1, PNS, 16)
  pf = _apply1(rows1, nxq, emp3_1, wp1,
               pmlp_gamma.reshape(1, 16), pmlp_beta.reshape(1, 16), st1)
  fw = _pad_q(jnp.tile(fc_W[0, :GRID**3 * 16].reshape(GRID**3, 16),
                       (B * NROIS, 1)))
  r2q = _fc(pf, fw, roi_features.reshape(B * NROIS, CROI),
            fc_W[0, GRID**3 * 16:].reshape(CROI, 1), fc_b.reshape(1, 1))

  rows2, emp2 = _sel_gather(nxq, r2q, xyzb, p2b, ptabb, NS)
  emp3_2 = emp2.reshape(MP, 1, 1)
  w1p = _padw(fmlp_W1)
  fst1 = _stats1(rows2, nxq, emp3_2, w1p, NS, 32)
  fst2 = _stats2(rows2, nxq, emp3_2, w1p, fmlp_g1.reshape(1, 32),
                 fmlp_b1.reshape(1, 32), fst1, fmlp_W2)
  tpd = jnp.asarray(temperature_decay, jnp.float32).reshape(1, 1)
  nf = _apply2(rows2, nxq, emp3_2, w1p, fmlp_g1.reshape(1, 32),
               fmlp_b1.reshape(1, 32), fst1, fmlp_W2, fmlp_g2.reshape(1, 64),
               fmlp_b2.reshape(1, 64), fst2, tpd)
  nfr = nf.reshape(B, MBP, 64)[:, :MB, :].reshape(M, 64)
  return new_xyz, jnp.transpose(nfr)[None]


# speedup vs baseline: 1.8474x; 1.8474x over previous
import functools

import jax
import jax.numpy as jnp
from jax import lax
from jax.experimental import pallas as pl

B = 2
NPTS = 4096
N = B * NPTS
NROIS = 64
GRID = 3
C = 32
CROI = 64
NS = 32
PNS = 16
RADIUS = 0.8
TEMP = 1.0
DIV = 2.0
MIN_R = 0.01
M = B * NROIS * GRID**3
MB = NROIS * GRID**3
MBP = 1792
MP = B * MBP
PW = 48
QBLK = 128
NBB = MBP // QBLK
NBLK = B * NBB
PCH = 512


def _sel_gather(nxq, r2q, xyzb, p2b, pth, ptm, ptl, ns):
  def f(nx_ref, r2_ref, xyz_ref, p2_ref, pth_ref, ptm_ref, ptl_ref,
        rows_ref, emp_ref):
    nx = nx_ref[...]
    r2 = r2_ref[...]
    pxyz = xyz_ref[0]
    dx = nx[:, 0:1] - pxyz[0:1, :]
    dy = nx[:, 1:2] - pxyz[1:2, :]
    dz = nx[:, 2:3] - pxyz[2:3, :]
    d2 = dx * dx + dy * dy + dz * dz
    vm = (d2 < r2).astype(jnp.float32)
    rio = lax.broadcasted_iota(jnp.int32, (PCH, PCH), 0)
    cio = lax.broadcasted_iota(jnp.int32, (PCH, PCH), 1)
    lt = (rio < cio).astype(jnp.float32)
    carry = jnp.zeros((QBLK, 1), jnp.float32)
    ranks = []
    for c in range(NPTS // PCH):
      vmc = vm[:, c * PCH:(c + 1) * PCH]
      ranks.append(jnp.dot(vmc, lt, preferred_element_type=jnp.float32) + carry)
      carry = carry + jnp.sum(vmc, axis=1, keepdims=True)
    rank = jnp.concatenate(ranks, axis=1)
    cnt = carry
    pth_ = pth_ref[0]
    ptm_ = ptm_ref[0]
    ptl_ = ptl_ref[0]
    gs = []
    g0 = None
    for s in range(ns):
      sel = vm * (rank == float(s)).astype(jnp.float32)
      g = (jnp.dot(sel, pth_, preferred_element_type=jnp.float32)
           + jnp.dot(sel, ptm_, preferred_element_type=jnp.float32)
           + jnp.dot(sel, ptl_, preferred_element_type=jnp.float32))
      if s == 0:
        g0 = g
      else:
        g = g + jnp.where(cnt <= float(s), 1.0, 0.0) * g0
      gs.append(g.reshape(QBLK, 1, PW))
    g3 = jnp.concatenate(gs, axis=1)
    rows_ref[...] = g3.reshape(QBLK * ns, PW)
    emp_ref[...] = jnp.where(cnt == 0.0, 1.0, 0.0)

  return pl.pallas_call(
      f, grid=(NBLK,),
      in_specs=[
          pl.BlockSpec((QBLK, 3), lambda j: (j, 0)),
          pl.BlockSpec((QBLK, 1), lambda j: (j, 0)),
          pl.BlockSpec((1, 3, NPTS), lambda j: (j // NBB, 0, 0)),
          pl.BlockSpec((1, 1, NPTS), lambda j: (j // NBB, 0, 0)),
          pl.BlockSpec((1, NPTS, PW), lambda j: (j // NBB, 0, 0)),
          pl.BlockSpec((1, NPTS, PW), lambda j: (j // NBB, 0, 0)),
          pl.BlockSpec((1, NPTS, PW), lambda j: (j // NBB, 0, 0)),
      ],
      out_specs=[
          pl.BlockSpec((QBLK * ns, PW), lambda j: (j, 0)),
          pl.BlockSpec((QBLK, 1), lambda j: (j, 0)),
      ],
      out_shape=[
          jax.ShapeDtypeStruct((MP * ns, PW), jnp.float32),
          jax.ShapeDtypeStruct((MP, 1), jnp.float32),
      ],
  )(nxq, r2q, xyzb, p2b, pth, ptm, ptl)


def _y3(rows_ref, nx_ref, emp_ref, w_ref, ns, co):
  rows3 = rows_ref[...].reshape(QBLK, ns, PW)
  gxyz = rows3[:, :, 0:3] - nx_ref[...][:, None, :]
  g3 = jnp.concatenate([gxyz, rows3[:, :, 3:]], axis=2)
  y2 = jnp.dot(g3.reshape(QBLK * ns, PW), w_ref[...].T,
               preferred_element_type=jnp.float32)
  y3 = y2.reshape(QBLK, ns, co)
  return jnp.where(emp_ref[...] > 0, 0.0, y3)


def _bn(y, st_ref, g_ref, b_ref, count, co):
  mean = (st_ref[0:1, 0:co] / count).reshape(1, 1, co)
  var = (st_ref[1:2, 0:co] / count).reshape(1, 1, co) - mean * mean
  inv = lax.rsqrt(var + 1e-5)
  z = (y - mean) * inv * g_ref[...].reshape(1, 1, co) + b_ref[...].reshape(1, 1, co)
  return jnp.maximum(z, 0.0)


def _stats_update(out_ref, y, co, j):
  @pl.when(j == 0)
  def _():
    out_ref[...] = jnp.zeros_like(out_ref)
  out_ref[0:1, 0:co] += jnp.sum(y, axis=(0, 1)).reshape(1, co)
  out_ref[1:2, 0:co] += jnp.sum(y * y, axis=(0, 1)).reshape(1, co)


def _specs(ns, co_w):
  rows_s = pl.BlockSpec((QBLK * ns, PW), lambda j: (j, 0))
  nx_s = pl.BlockSpec((QBLK, 3), lambda j: (j, 0))
  emp_s = pl.BlockSpec((QBLK, 1, 1), lambda j: (j, 0, 0))
  w_s = pl.BlockSpec((co_w, PW), lambda j: (0, 0))
  st_s = pl.BlockSpec((8, 128), lambda j: (0, 0))
  return rows_s, nx_s, emp_s, w_s, st_s


def _stats1(rows, nx, emp3, wp, ns, co):
  rows_s, nx_s, emp_s, w_s, st_s = _specs(ns, co)

  def f(rows_ref, nx_ref, emp_ref, w_ref, out_ref):
    j = pl.program_id(0)
    y = _y3(rows_ref, nx_ref, emp_ref, w_ref, ns, co)
    _stats_update(out_ref, y, co, j)

  return pl.pallas_call(
      f, grid=(NBLK,),
      in_specs=[rows_s, nx_s, emp_s, w_s],
      out_specs=st_s,
      out_shape=jax.ShapeDtypeStruct((8, 128), jnp.float32),
  )(rows, nx, emp3, wp)


def _apply1(rows, nx, emp3, wp, g, b, st):
  rows_s, nx_s, emp_s, w_s, st_s = _specs(PNS, 16)
  gb_s = pl.BlockSpec((1, 16), lambda j: (0, 0))

  def f(rows_ref, nx_ref, emp_ref, w_ref, g_ref, b_ref, st_ref, out_ref):
    y = _y3(rows_ref, nx_ref, emp_ref, w_ref, PNS, 16)
    z = _bn(y, st_ref, g_ref, b_ref, float(M * PNS), 16)
    out_ref[...] = jnp.max(z, axis=1)

  return pl.pallas_call(
      f, grid=(NBLK,),
      in_specs=[rows_s, nx_s, emp_s, w_s, gb_s, gb_s, st_s],
      out_specs=pl.BlockSpec((QBLK, 16), lambda j: (j, 0)),
      out_shape=jax.ShapeDtypeStruct((MP, 16), jnp.float32),
  )(rows, nx, emp3, wp, g, b, st)


def _bfr(x):
  xi = lax.bitcast_convert_type(x, jnp.uint32)
  xr = (xi + jnp.uint32(0x7FFF) + ((xi >> 16) & jnp.uint32(1))) & jnp.uint32(0xFFFF0000)
  return lax.bitcast_convert_type(xr, jnp.float32)


def _fc(pf, fwrep, rf, w2, bias):
  def f(pf_ref, fw_ref, rf_ref, w2_ref, b_ref, out_ref):
    a = jnp.sum(_bfr(pf_ref[...]) * _bfr(fw_ref[...]), axis=1, keepdims=True)
    mio = lax.broadcasted_iota(jnp.int32, (B * NROIS, MP), 1)
    rio = lax.broadcasted_iota(jnp.int32, (B * NROIS, MP), 0)
    roi_of = (mio // MBP) * NROIS + (mio % MBP) // (GRID**3)
    vrow = (mio % MBP) < MB
    S = ((roi_of == rio) & vrow).astype(jnp.float32)
    resid = jnp.dot(S, a, preferred_element_type=jnp.float32,
                    precision=lax.Precision.HIGHEST)
    resid += jnp.dot(_bfr(rf_ref[...]), _bfr(w2_ref[...]),
                     preferred_element_type=jnp.float32,
                     precision=lax.Precision.HIGHEST)
    r = jnp.maximum((resid + b_ref[0, 0]) / DIV + RADIUS, MIN_R)
    qio = lax.broadcasted_iota(jnp.int32, (MP, B * NROIS), 0)
    cio = lax.broadcasted_iota(jnp.int32, (MP, B * NROIS), 1)
    roi_oq = (qio // MBP) * NROIS + (qio % MBP) // (GRID**3)
    vq = (qio % MBP) < MB
    T = ((roi_oq == cio) & vq).astype(jnp.float32)
    r2q = jnp.dot(T, r * r, preferred_element_type=jnp.float32,
                  precision=lax.Precision.HIGHEST)
    qv = lax.broadcasted_iota(jnp.int32, (MP, 1), 0)
    out_ref[...] = jnp.where(qv % MBP < MB, r2q, -1.0)

  def full(s):
    return pl.BlockSpec(s, lambda: tuple(0 for _ in s))

  return pl.pallas_call(
      f,
      in_specs=[full((MP, 16)), full((MP, 16)), full((B * NROIS, CROI)),
                full((CROI, 1)), full((1, 1))],
      out_specs=full((MP, 1)),
      out_shape=jax.ShapeDtypeStruct((MP, 1), jnp.float32),
  )(pf, fwrep, rf, w2, bias)


def _stats2(rows, nx, emp3, w1p, g1, b1, st1, w2):
  rows_s, nx_s, emp_s, w_s, st_s = _specs(NS, 32)
  gb_s = pl.BlockSpec((1, 32), lambda j: (0, 0))
  w2_s = pl.BlockSpec((64, 32), lambda j: (0, 0))

  def f(rows_ref, nx_ref, emp_ref, w1_ref, g1_ref, b1_ref, st1_ref, w2_ref,
        out_ref):
    j = pl.program_id(0)
    y1 = _y3(rows_ref, nx_ref, emp_ref, w1_ref, NS, 32)
    z1 = _bn(y1, st1_ref, g1_ref, b1_ref, float(M * NS), 32)
    y2 = jnp.dot(z1.reshape(QBLK * NS, 32), w2_ref[...].T,
                 preferred_element_type=jnp.float32)
    y2 = y2.reshape(QBLK, NS, 64)
    _stats_update(out_ref, y2, 64, j)

  return pl.pallas_call(
      f, grid=(NBLK,),
      in_specs=[rows_s, nx_s, emp_s, w_s, gb_s, gb_s, st_s, w2_s],
      out_specs=st_s,
      out_shape=jax.ShapeDtypeStruct((8, 128), jnp.float32),
  )(rows, nx, emp3, w1p, g1, b1, st1, w2)


def _apply2(rows, nx, emp3, w1p, g1, b1, st1, w2, g2, b2, st2, tpd):
  rows_s, nx_s, emp_s, w_s, st_s = _specs(NS, 32)
  gb1_s = pl.BlockSpec((1, 32), lambda j: (0, 0))
  gb2_s = pl.BlockSpec((1, 64), lambda j: (0, 0))
  w2_s = pl.BlockSpec((64, 32), lambda j: (0, 0))
  tpd_s = pl.BlockSpec((1, 1), lambda j: (0, 0))

  def f(rows_ref, nx_ref, emp_ref, w1_ref, g1_ref, b1_ref, st1_ref, w2_ref,
        g2_ref, b2_ref, st2_ref, tpd_ref, out_ref):
    y1 = _y3(rows_ref, nx_ref, emp_ref, w1_ref, NS, 32)
    z1 = _bn(y1, st1_ref, g1_ref, b1_ref, float(M * NS), 32)
    y2 = jnp.dot(z1.reshape(QBLK * NS, 32), w2_ref[...].T,
                 preferred_element_type=jnp.float32).reshape(QBLK, NS, 64)
    z2 = _bn(y2, st2_ref, g2_ref, b2_ref, float(M * NS), 64)
    rows3 = rows_ref[...].reshape(QBLK, NS, PW)
    gxyz = rows3[:, :, 0:3] - nx_ref[...][:, None, :]
    d2v = jnp.sum(gxyz * gxyz, axis=2, keepdims=True)
    dist = jnp.sqrt(d2v + 1e-12)
    logit = -dist / (TEMP * tpd_ref[0, 0])
    mx = jnp.max(logit, axis=1, keepdims=True)
    e = jnp.exp(logit - mx)
    w = e / jnp.sum(e, axis=1, keepdims=True)
    w = jnp.where(emp_ref[...] > 0, 0.0, w)
    out_ref[...] = jnp.max(w * z2, axis=1)

  return pl.pallas_call(
      f, grid=(NBLK,),
      in_specs=[rows_s, nx_s, emp_s, w_s, gb1_s, gb1_s, st_s, w2_s,
                gb2_s, gb2_s, st_s, tpd_s],
      out_specs=pl.BlockSpec((QBLK, 64), lambda j: (j, 0)),
      out_shape=jax.ShapeDtypeStruct((MP, 64), jnp.float32),
  )(rows, nx, emp3, w1p, g1, b1, st1, w2, g2, b2, st2, tpd)


def _padw(w):
  return jnp.concatenate(
      [w, jnp.zeros((w.shape[0], PW - w.shape[1]), jnp.float32)], axis=1)


def _pad_q(x):
  d = x.shape[1]
  xb = x.reshape(B, MB, d)
  pad = jnp.zeros((B, MBP - MB, d), x.dtype)
  return jnp.concatenate([xb, pad], axis=1).reshape(MP, d)


def kernel(xyz, xyz_batch_cnt, rois, roi_features, features,
           temperature_decay, pmlp_W, pmlp_gamma, pmlp_beta, fc_W, fc_b,
           fmlp_W1, fmlp_g1, fmlp_b1, fmlp_W2, fmlp_g2, fmlp_b2):
  new_xyz = rois.reshape(-1, 3)
  nxq = _pad_q(new_xyz)
  xyzb = xyz.reshape(B, NPTS, 3).transpose(0, 2, 1)
  p2b = jnp.sum(xyzb * xyzb, axis=1, keepdims=True)
  ptab = jnp.concatenate(
      [xyz, features, jnp.zeros((N, PW - 3 - C), jnp.float32)],
      axis=1).reshape(B, NPTS, PW)
  pth = _bfr(ptab)
  rem = ptab - pth
  ptm = _bfr(rem)
  ptl = rem - ptm
  qv = jnp.arange(MP)
  r2p1 = jnp.where(qv % MBP < MB, jnp.float32(RADIUS**2), -1.0).reshape(MP, 1)

  rows1, emp1 = _sel_gather(nxq, r2p1, xyzb, p2b, pth, ptm, ptl, PNS)
  emp3_1 = emp1.reshape(MP, 1, 1)
  wp1 = _padw(pmlp_W)
  st1 = _stats1(rows1, nxq, emp3_1, wp1, PNS, 16)
  pf = _apply1(rows1, nxq, emp3_1, wp1,
               pmlp_gamma.reshape(1, 16), pmlp_beta.reshape(1, 16), st1)
  fw = _pad_q(jnp.tile(fc_W[0, :GRID**3 * 16].reshape(GRID**3, 16),
                       (B * NROIS, 1)))
  r2q = _fc(pf, fw, roi_features.reshape(B * NROIS, CROI),
            fc_W[0, GRID**3 * 16:].reshape(CROI, 1), fc_b.reshape(1, 1))

  rows2, emp2 = _sel_gather(nxq, r2q, xyzb, p2b, pth, ptm, ptl, NS)
  emp3_2 = emp2.reshape(MP, 1, 1)
  w1p = _padw(fmlp_W1)
  fst1 = _stats1(rows2, nxq, emp3_2, w1p, NS, 32)
  fst2 = _stats2(rows2, nxq, emp3_2, w1p, fmlp_g1.reshape(1, 32),
                 fmlp_b1.reshape(1, 32), fst1, fmlp_W2)
  tpd = jnp.asarray(temperature_decay, jnp.float32).reshape(1, 1)
  nf = _apply2(rows2, nxq, emp3_2, w1p, fmlp_g1.reshape(1, 32),
               fmlp_b1.reshape(1, 32), fst1, fmlp_W2, fmlp_g2.reshape(1, 64),
               fmlp_b2.reshape(1, 64), fst2, tpd)
  nfr = nf.reshape(B, MBP, 64)[:, :MB, :].reshape(M, 64)
  return new_xyz, jnp.transpose(nfr)[None]
